# baseline (speedup 1.0000x reference)
import functools

import jax
import jax.numpy as jnp
from jax import lax
from jax.experimental import pallas as pl
from jax.experimental.pallas import tpu as pltpu
from jax.experimental.pallas import tpu_sc as plsc

_NC = 2
_NS = 16
_NW = _NC * _NS
_L = 16

_C = 128


def kernel(feat, batch_ids, glob_feat):
    n = feat.shape[0]
    b, d = glob_feat.shape
    idx = batch_ids.astype(jnp.int32)
    glob_flat = glob_feat.reshape(-1)

    num_chunks = -(-n // _C)
    trips = -(-num_chunks // _NW)
    span = trips * _C
    idx_base_max = n - span
    ncg = d // _L
    n_search = max(1, (span - 1).bit_length())

    mesh = plsc.VectorSubcoreMesh(
        core_axis_name="c", subcore_axis_name="s",
        num_cores=_NC, num_subcores=_NS,
    )

    @functools.partial(
        pl.kernel,
        out_type=jax.ShapeDtypeStruct((n * d,), jnp.float32),
        mesh=mesh,
        scratch_types=[
            pltpu.VMEM((span + _L,), jnp.int32),
            pltpu.VMEM((b * d,), jnp.float32),
            pltpu.VMEM((_C * d,), jnp.float32),
            pltpu.SMEM((b + 1,), jnp.int32),
            pltpu.SemaphoreType.DMA,
        ],
    )
    def bcast(idx_hbm, glob_hbm, out_hbm, idx_v, glob_v, rows_v, bnd_s,
              sem_w):
        wid = lax.axis_index("s") * _NC + lax.axis_index("c")
        r0 = ((wid * num_chunks) // _NW) * _C
        r1 = jnp.minimum((((wid + 1) * num_chunks) // _NW) * _C, n)
        idx_base = jnp.minimum(r0, idx_base_max)

        cp_g = pltpu.async_copy(glob_hbm, glob_v, sem_w)
        cp_i = pltpu.async_copy(idx_hbm.at[pl.ds(idx_base, span)],
                                idx_v.at[pl.ds(0, span)], sem_w)
        cp_g.wait()
        cp_i.wait()

        def search(v, carry):
            def step(_, lh):
                lo, hi = lh
                active = lo < hi
                mid = (lo + hi) // 2
                val = idx_v[pl.ds(mid - idx_base, _L)][0]
                go_right = jnp.logical_and(active, val < v)
                lo = jnp.where(go_right, mid + 1, lo)
                hi = jnp.where(
                    jnp.logical_and(active, jnp.logical_not(val < v)),
                    mid, hi)
                return lo, hi
            lo, _ = lax.fori_loop(0, n_search, step, (r0, r1))
            bnd_s[v] = lo
            return carry
        lax.fori_loop(0, b, search, 0)
        bnd_s[b] = r1

        def fill_uniform(bid):
            row = [glob_v[pl.ds(bid * d + k * _L, _L)] for k in range(ncg)]

            def store_row(r, carry):
                for k in range(ncg):
                    rows_v[pl.ds(r * d + k * _L, _L)] = row[k]
                return carry
            lax.fori_loop(0, _C, store_row, 0)

        def drain(k):
            def wait_one(_, carry):
                pltpu.make_async_copy(
                    rows_v, out_hbm.at[pl.ds(0, _C * d)], sem_w).wait()
                return carry
            lax.fori_loop(0, k, wait_one, 0)

        def run_body(v, n_out):
            start = bnd_s[v]
            end = bnd_s[v + 1]
            m = end - start
            n_full = m >> (_C.bit_length() - 1)
            rem = m & (_C - 1)
            active = m > 0
            has_tail = jnp.logical_and(rem > 0, m >= _C)

            @pl.when(active)
            def _():
                drain(n_out)
                fill_uniform(v)

                def wr(k, carry):
                    pltpu.async_copy(
                        rows_v,
                        out_hbm.at[pl.ds((start + k * _C) * d, _C * d)],
                        sem_w)
                    return carry
                lax.fori_loop(0, n_full, wr, 0)

                @pl.when(has_tail)
                def _():
                    pltpu.async_copy(
                        rows_v, out_hbm.at[pl.ds((end - _C) * d, _C * d)],
                        sem_w)

                @pl.when(m < _C)
                def _():
                    off = start
                    for sz in [_C >> (s + 1)
                               for s in range(_C.bit_length() - 1)]:
                        hit = (m & sz) != 0

                        @pl.when(hit)
                        def _(off=off, sz=sz):
                            pltpu.sync_copy(
                                rows_v.at[pl.ds(0, sz * d)],
                                out_hbm.at[pl.ds(off * d, sz * d)])
                        off = off + jnp.where(hit, sz, 0)

            return jnp.where(active,
                             n_full + jnp.where(has_tail, 1, 0),
                             n_out)

        n_left = lax.fori_loop(0, b, run_body, jnp.int32(0))
        drain(n_left)

    out = bcast(idx, glob_flat)
    return out.reshape(n, d)


# speedup vs baseline: 1.0382x; 1.0382x over previous
import functools

import jax
import jax.numpy as jnp
from jax import lax
from jax.experimental import pallas as pl
from jax.experimental.pallas import tpu as pltpu
from jax.experimental.pallas import tpu_sc as plsc

_NC = 2
_NS = 16
_NW = _NC * _NS
_L = 16

_C = 32


def kernel(feat, batch_ids, glob_feat):
    n = feat.shape[0]
    b, d = glob_feat.shape
    idx = batch_ids.astype(jnp.int32)
    glob_flat = glob_feat.reshape(-1)

    num_chunks = -(-n // _C)
    trips = -(-num_chunks // _NW)
    span = trips * _C
    idx_base_max = n - span
    ncg = d // _L
    n_search = max(1, (span - 1).bit_length())

    mesh = plsc.VectorSubcoreMesh(
        core_axis_name="c", subcore_axis_name="s",
        num_cores=_NC, num_subcores=_NS,
    )

    @functools.partial(
        pl.kernel,
        out_type=jax.ShapeDtypeStruct((n * d,), jnp.float32),
        mesh=mesh,
        scratch_types=[
            pltpu.VMEM((span + _L,), jnp.int32),
            pltpu.VMEM((b * d,), jnp.float32),
            pltpu.VMEM((_C * d,), jnp.float32),
            pltpu.SMEM((b + 1,), jnp.int32),
            pltpu.SemaphoreType.DMA,
        ],
    )
    def bcast(idx_hbm, glob_hbm, out_hbm, idx_v, glob_v, rows_v, bnd_s,
              sem_w):
        wid = lax.axis_index("s") * _NC + lax.axis_index("c")
        r0 = ((wid * num_chunks) // _NW) * _C
        r1 = jnp.minimum((((wid + 1) * num_chunks) // _NW) * _C, n)
        idx_base = jnp.minimum(r0, idx_base_max)

        cp_g = pltpu.async_copy(glob_hbm, glob_v, sem_w)
        cp_i = pltpu.async_copy(idx_hbm.at[pl.ds(idx_base, span)],
                                idx_v.at[pl.ds(0, span)], sem_w)
        cp_g.wait()
        cp_i.wait()

        def search(v, carry):
            def step(_, lh):
                lo, hi = lh
                active = lo < hi
                mid = (lo + hi) // 2
                val = idx_v[pl.ds(mid - idx_base, _L)][0]
                go_right = jnp.logical_and(active, val < v)
                lo = jnp.where(go_right, mid + 1, lo)
                hi = jnp.where(
                    jnp.logical_and(active, jnp.logical_not(val < v)),
                    mid, hi)
                return lo, hi
            lo, _ = lax.fori_loop(0, n_search, step, (r0, r1))
            bnd_s[v] = lo
            return carry
        lax.fori_loop(0, b, search, 0)
        bnd_s[b] = r1

        def fill_uniform(bid):
            row = [glob_v[pl.ds(bid * d + k * _L, _L)] for k in range(ncg)]

            def store_row(r, carry):
                for k in range(ncg):
                    rows_v[pl.ds(r * d + k * _L, _L)] = row[k]
                return carry
            lax.fori_loop(0, _C, store_row, 0)

        def drain(k):
            def wait_one(_, carry):
                pltpu.make_async_copy(
                    rows_v, out_hbm.at[pl.ds(0, _C * d)], sem_w).wait()
                return carry
            lax.fori_loop(0, k, wait_one, 0)

        def run_body(v, n_out):
            start = bnd_s[v]
            end = bnd_s[v + 1]
            m = end - start
            n_full = m >> (_C.bit_length() - 1)
            rem = m & (_C - 1)
            active = m > 0
            has_tail = jnp.logical_and(rem > 0, m >= _C)

            @pl.when(active)
            def _():
                drain(n_out)
                fill_uniform(v)

                def wr(k, carry):
                    pltpu.async_copy(
                        rows_v,
                        out_hbm.at[pl.ds((start + k * _C) * d, _C * d)],
                        sem_w)
                    return carry
                lax.fori_loop(0, n_full, wr, 0)

                @pl.when(has_tail)
                def _():
                    pltpu.async_copy(
                        rows_v, out_hbm.at[pl.ds((end - _C) * d, _C * d)],
                        sem_w)

                @pl.when(m < _C)
                def _():
                    off = start
                    for sz in [_C >> (s + 1)
                               for s in range(_C.bit_length() - 1)]:
                        hit = (m & sz) != 0

                        @pl.when(hit)
                        def _(off=off, sz=sz):
                            pltpu.sync_copy(
                                rows_v.at[pl.ds(0, sz * d)],
                                out_hbm.at[pl.ds(off * d, sz * d)])
                        off = off + jnp.where(hit, sz, 0)

            return jnp.where(active,
                             n_full + jnp.where(has_tail, 1, 0),
                             n_out)

        n_left = lax.fori_loop(0, b, run_body, jnp.int32(0))
        drain(n_left)

    out = bcast(idx, glob_flat)
    return out.reshape(n, d)


# speedup vs baseline: 1.0384x; 1.0002x over previous
import functools

import jax
import jax.numpy as jnp
from jax import lax
from jax.experimental import pallas as pl
from jax.experimental.pallas import tpu as pltpu
from jax.experimental.pallas import tpu_sc as plsc

_NC = 2
_NS = 16
_NW = _NC * _NS
_L = 16

_C = 16


def kernel(feat, batch_ids, glob_feat):
    n = feat.shape[0]
    b, d = glob_feat.shape
    idx = batch_ids.astype(jnp.int32)
    glob_flat = glob_feat.reshape(-1)

    num_chunks = -(-n // _C)
    trips = -(-num_chunks // _NW)
    span = trips * _C
    idx_base_max = n - span
    ncg = d // _L
    n_search = max(1, (span - 1).bit_length())

    mesh = plsc.VectorSubcoreMesh(
        core_axis_name="c", subcore_axis_name="s",
        num_cores=_NC, num_subcores=_NS,
    )

    @functools.partial(
        pl.kernel,
        out_type=jax.ShapeDtypeStruct((n * d,), jnp.float32),
        mesh=mesh,
        scratch_types=[
            pltpu.VMEM((span + _L,), jnp.int32),
            pltpu.VMEM((b * d,), jnp.float32),
            pltpu.VMEM((_C * d,), jnp.float32),
            pltpu.SMEM((b + 1,), jnp.int32),
            pltpu.SemaphoreType.DMA,
        ],
    )
    def bcast(idx_hbm, glob_hbm, out_hbm, idx_v, glob_v, rows_v, bnd_s,
              sem_w):
        wid = lax.axis_index("s") * _NC + lax.axis_index("c")
        r0 = ((wid * num_chunks) // _NW) * _C
        r1 = jnp.minimum((((wid + 1) * num_chunks) // _NW) * _C, n)
        idx_base = jnp.minimum(r0, idx_base_max)

        cp_g = pltpu.async_copy(glob_hbm, glob_v, sem_w)
        cp_i = pltpu.async_copy(idx_hbm.at[pl.ds(idx_base, span)],
                                idx_v.at[pl.ds(0, span)], sem_w)
        cp_g.wait()
        cp_i.wait()

        def search(v, carry):
            def step(_, lh):
                lo, hi = lh
                active = lo < hi
                mid = (lo + hi) // 2
                val = idx_v[pl.ds(mid - idx_base, _L)][0]
                go_right = jnp.logical_and(active, val < v)
                lo = jnp.where(go_right, mid + 1, lo)
                hi = jnp.where(
                    jnp.logical_and(active, jnp.logical_not(val < v)),
                    mid, hi)
                return lo, hi
            lo, _ = lax.fori_loop(0, n_search, step, (r0, r1))
            bnd_s[v] = lo
            return carry
        lax.fori_loop(0, b, search, 0)
        bnd_s[b] = r1

        def fill_uniform(bid):
            row = [glob_v[pl.ds(bid * d + k * _L, _L)] for k in range(ncg)]

            def store_row(r, carry):
                for k in range(ncg):
                    rows_v[pl.ds(r * d + k * _L, _L)] = row[k]
                return carry
            lax.fori_loop(0, _C, store_row, 0)

        def drain(k):
            def wait_one(_, carry):
                pltpu.make_async_copy(
                    rows_v, out_hbm.at[pl.ds(0, _C * d)], sem_w).wait()
                return carry
            lax.fori_loop(0, k, wait_one, 0)

        def run_body(v, n_out):
            start = bnd_s[v]
            end = bnd_s[v + 1]
            m = end - start
            n_full = m >> (_C.bit_length() - 1)
            rem = m & (_C - 1)
            active = m > 0
            has_tail = jnp.logical_and(rem > 0, m >= _C)

            @pl.when(active)
            def _():
                drain(n_out)
                fill_uniform(v)

                def wr(k, carry):
                    pltpu.async_copy(
                        rows_v,
                        out_hbm.at[pl.ds((start + k * _C) * d, _C * d)],
                        sem_w)
                    return carry
                lax.fori_loop(0, n_full, wr, 0)

                @pl.when(has_tail)
                def _():
                    pltpu.async_copy(
                        rows_v, out_hbm.at[pl.ds((end - _C) * d, _C * d)],
                        sem_w)

                @pl.when(m < _C)
                def _():
                    off = start
                    for sz in [_C >> (s + 1)
                               for s in range(_C.bit_length() - 1)]:
                        hit = (m & sz) != 0

                        @pl.when(hit)
                        def _(off=off, sz=sz):
                            pltpu.sync_copy(
                                rows_v.at[pl.ds(0, sz * d)],
                                out_hbm.at[pl.ds(off * d, sz * d)])
                        off = off + jnp.where(hit, sz, 0)

            return jnp.where(active,
                             n_full + jnp.where(has_tail, 1, 0),
                             n_out)

        n_left = lax.fori_loop(0, b, run_body, jnp.int32(0))
        drain(n_left)

    out = bcast(idx, glob_flat)
    return out.reshape(n, d)


# speedup vs baseline: 1.1364x; 1.0944x over previous
import functools

import jax
import jax.numpy as jnp
from jax import lax
from jax.experimental import pallas as pl
from jax.experimental.pallas import tpu as pltpu
from jax.experimental.pallas import tpu_sc as plsc

_NC = 2
_NS = 16
_NW = _NC * _NS
_L = 16

_C = 16


def kernel(feat, batch_ids, glob_feat):
    n = feat.shape[0]
    b, d = glob_feat.shape
    idx = batch_ids.astype(jnp.int32)
    glob_flat = glob_feat.reshape(-1)

    num_chunks = -(-n // _C)
    trips = -(-num_chunks // _NW)
    span = trips * _C
    idx_base_max = n - span
    ncg = d // _L
    n_search = max(1, (span - 1).bit_length())

    mesh = plsc.VectorSubcoreMesh(
        core_axis_name="c", subcore_axis_name="s",
        num_cores=_NC, num_subcores=_NS,
    )

    @functools.partial(
        pl.kernel,
        out_type=jax.ShapeDtypeStruct((n * d,), jnp.float32),
        mesh=mesh,
        scratch_types=[
            pltpu.VMEM((span + _L,), jnp.int32),
            pltpu.VMEM((b * d,), jnp.float32),
            pltpu.VMEM((_C * d,), jnp.float32),
            pltpu.SMEM((b + 1,), jnp.int32),
            pltpu.SemaphoreType.DMA,
        ],
    )
    def bcast(idx_hbm, glob_hbm, out_hbm, idx_v, glob_v, rows_v, bnd_s,
              sem_w):
        wid = lax.axis_index("s") * _NC + lax.axis_index("c")
        r0 = ((wid * num_chunks) // _NW) * _C
        r1 = jnp.minimum((((wid + 1) * num_chunks) // _NW) * _C, n)
        idx_base = jnp.minimum(r0, idx_base_max)

        cp_g = pltpu.async_copy(glob_hbm, glob_v, sem_w)
        cp_i = pltpu.async_copy(idx_hbm.at[pl.ds(idx_base, span)],
                                idx_v.at[pl.ds(0, span)], sem_w)
        cp_g.wait()
        cp_i.wait()

        v_first = idx_v[pl.ds(r0 - idx_base, _L)][0]
        v_last = idx_v[pl.ds(r1 - 1 - idx_base, _L)][0]

        def search(v, carry):
            need = jnp.logical_and(v > v_first, v <= v_last)

            def do_search():
                def step(_, lh):
                    lo, hi = lh
                    active = lo < hi
                    mid = (lo + hi) // 2
                    val = idx_v[pl.ds(mid - idx_base, _L)][0]
                    go_right = jnp.logical_and(active, val < v)
                    lo = jnp.where(go_right, mid + 1, lo)
                    hi = jnp.where(
                        jnp.logical_and(active, jnp.logical_not(val < v)),
                        mid, hi)
                    return lo, hi
                lo, _ = lax.fori_loop(0, n_search, step, (r0, r1))
                return lo

            bnd_s[v] = lax.cond(
                need, do_search,
                lambda: jnp.where(v <= v_first, r0, r1))
            return carry
        lax.fori_loop(0, b, search, 0)
        bnd_s[b] = r1

        def fill_uniform(bid):
            row = [glob_v[pl.ds(bid * d + k * _L, _L)] for k in range(ncg)]

            def store_row(r, carry):
                for k in range(ncg):
                    rows_v[pl.ds(r * d + k * _L, _L)] = row[k]
                return carry
            lax.fori_loop(0, _C, store_row, 0)

        def drain(k):
            def wait_one(_, carry):
                pltpu.make_async_copy(
                    rows_v, out_hbm.at[pl.ds(0, _C * d)], sem_w).wait()
                return carry
            lax.fori_loop(0, k, wait_one, 0)

        def run_body(v, n_out):
            start = bnd_s[v]
            end = bnd_s[v + 1]
            m = end - start
            n_full = m >> (_C.bit_length() - 1)
            rem = m & (_C - 1)
            active = m > 0
            has_tail = jnp.logical_and(rem > 0, m >= _C)

            @pl.when(active)
            def _():
                drain(n_out)
                fill_uniform(v)

                def wr(k, carry):
                    pltpu.async_copy(
                        rows_v,
                        out_hbm.at[pl.ds((start + k * _C) * d, _C * d)],
                        sem_w)
                    return carry
                lax.fori_loop(0, n_full, wr, 0)

                @pl.when(has_tail)
                def _():
                    pltpu.async_copy(
                        rows_v, out_hbm.at[pl.ds((end - _C) * d, _C * d)],
                        sem_w)

                @pl.when(m < _C)
                def _():
                    off = start
                    for sz in [_C >> (s + 1)
                               for s in range(_C.bit_length() - 1)]:
                        hit = (m & sz) != 0

                        @pl.when(hit)
                        def _(off=off, sz=sz):
                            pltpu.sync_copy(
                                rows_v.at[pl.ds(0, sz * d)],
                                out_hbm.at[pl.ds(off * d, sz * d)])
                        off = off + jnp.where(hit, sz, 0)

            return jnp.where(active,
                             n_full + jnp.where(has_tail, 1, 0),
                             n_out)

        n_left = lax.fori_loop(0, b, run_body, jnp.int32(0))
        drain(n_left)

    out = bcast(idx, glob_flat)
    return out.reshape(n, d)
